# baseline (speedup 1.0000x reference)
import functools

import jax
import jax.numpy as jnp
from jax import lax
from jax.experimental import pallas as pl
from jax.experimental.pallas import tpu as pltpu
from jax.experimental.pallas import tpu_sc as plsc

N = 1_000_000
V = 100_000
L = 524_288
MAXLEN = 8

VP = 100_096
NW = 32
CHUNK = 4096
NP = 1 << 20
CHUNKS_PER_TILE = NP // (NW * CHUNK)
GROUPS = CHUNK // 16
PADL = L + 256
TRASH = L
ZSLICE = L // 16
WSLICE = PADL // 16

_mesh = plsc.VectorSubcoreMesh(core_axis_name="c", subcore_axis_name="s")
_sc_params = pltpu.CompilerParams(use_tc_tiling_on_sc=False,
                                  needs_layout_passes=False)


@functools.partial(
    pl.kernel,
    out_type=(jax.ShapeDtypeStruct((2, PADL), jnp.float32),
              jax.ShapeDtypeStruct((2, PADL), jnp.float32)),
    mesh=_mesh,
    scratch_types=[
        pltpu.VMEM((CHUNK,), jnp.int32),
        pltpu.VMEM((CHUNK,), jnp.int32),
        pltpu.VMEM((CHUNK,), jnp.int32),
        pltpu.VMEM((CHUNK,), jnp.int32),
        pltpu.VMEM((CHUNK,), jnp.float32),
        pltpu.VMEM((CHUNK,), jnp.float32),
        pltpu.VMEM((CHUNK,), jnp.int32),
        pltpu.VMEM((CHUNK,), jnp.int32),
        pltpu.VMEM((CHUNK,), jnp.float32),
        pltpu.VMEM((CHUNK,), jnp.float32),
        pltpu.VMEM((2 * CHUNK,), jnp.float32),
        pltpu.VMEM_SHARED((PADL,), jnp.float32),
        pltpu.VMEM_SHARED((PADL,), jnp.float32),
        pltpu.SemaphoreType.DMA,
    ],
    compiler_params=_sc_params,
)
def _scatter_diff(sp_hbm, ti_hbm, tl_hbm, nd_hbm, par_hbm, osum_hbm, ocnt_hbm,
                  sp_v, ti_v, tl_v, nd_v, vals_v, vneg_v, ia_v, ib_v,
                  cpos_v, cneg_v, zf_v, asum_sh, acnt_sh, sem):
    c = lax.axis_index("c")
    s = lax.axis_index("s")
    wid = s * 2 + c

    ones_f = jnp.full((16,), 1.0, jnp.float32)
    zero_f = jnp.zeros((16,), jnp.float32)

    def _fill_cnt(i, _):
        cpos_v[pl.ds(i * 16, 16)] = ones_f
        cneg_v[pl.ds(i * 16, 16)] = -ones_f
        return 0
    lax.fori_loop(0, CHUNK // 16, _fill_cnt, 0)

    def _fill_zf(i, _):
        zf_v[pl.ds(i * 16, 16)] = zero_f
        return 0
    lax.fori_loop(0, 2 * CHUNK // 16, _fill_zf, 0)

    for k in range(ZSLICE // (2 * CHUNK)):
        pltpu.sync_copy(zf_v, asum_sh.at[pl.ds(s * ZSLICE + k * 2 * CHUNK,
                                               2 * CHUNK)])
    for k in range(ZSLICE // (2 * CHUNK)):
        pltpu.sync_copy(zf_v, acnt_sh.at[pl.ds(s * ZSLICE + k * 2 * CHUNK,
                                               2 * CHUNK)])
    @pl.when(s == 0)
    def _():
        pltpu.sync_copy(zf_v.at[pl.ds(0, 256)], asum_sh.at[pl.ds(L, 256)])
        pltpu.sync_copy(zf_v.at[pl.ds(0, 256)], acnt_sh.at[pl.ds(L, 256)])

    plsc.subcore_barrier()

    for k in range(CHUNKS_PER_TILE):
        gbase = (wid * CHUNKS_PER_TILE + k) * CHUNK
        pltpu.sync_copy(sp_hbm.at[pl.ds(gbase, CHUNK)], sp_v)
        pltpu.sync_copy(ti_hbm.at[pl.ds(gbase, CHUNK)], ti_v)
        pltpu.sync_copy(tl_hbm.at[pl.ds(gbase, CHUNK)], tl_v)
        pltpu.sync_copy(nd_hbm.at[pl.ds(gbase, CHUNK)], nd_v)
        pltpu.async_copy(par_hbm.at[ti_v], vals_v, sem).wait()

        def _group(g, _):
            st = sp_v[pl.ds(g * 16, 16)]
            ln = tl_v[pl.ds(g * 16, 16)]
            nd = nd_v[pl.ds(g * 16, 16)] > 0
            val = vals_v[pl.ds(g * 16, 16)]
            ia_v[pl.ds(g * 16, 16)] = jnp.where(nd, st, TRASH)
            ib_v[pl.ds(g * 16, 16)] = jnp.where(nd, st + ln, TRASH)
            vneg_v[pl.ds(g * 16, 16)] = -val
            return 0
        lax.fori_loop(0, GROUPS, _group, 0)

        pltpu.sync_copy(vals_v, asum_sh.at[ia_v], add=True)
        pltpu.sync_copy(vneg_v, asum_sh.at[ib_v], add=True)
        pltpu.sync_copy(cpos_v, acnt_sh.at[ia_v], add=True)
        pltpu.sync_copy(cneg_v, acnt_sh.at[ib_v], add=True)

    plsc.subcore_barrier()
    pltpu.sync_copy(asum_sh.at[pl.ds(s * WSLICE, WSLICE)],
                    osum_hbm.at[c, pl.ds(s * WSLICE, WSLICE)])
    pltpu.sync_copy(acnt_sh.at[pl.ds(s * WSLICE, WSLICE)],
                    ocnt_hbm.at[c, pl.ds(s * WSLICE, WSLICE)])


def _mm(a, b):
    return lax.dot_general(a, b, (((1,), (0,)), ((), ())),
                           precision=lax.Precision.HIGHEST,
                           preferred_element_type=jnp.float32)


def _cumsum_rowmajor(x, t128, s128):
    y = _mm(x, t128)
    blocks = []
    run = jnp.float32(0.0)
    for b in range(32):
        yb = y[b * 128:(b + 1) * 128, :]
        tb = yb[:, 127:128]
        eb = _mm(s128, tb)
        blocks.append(yb + (eb + run))
        run = run + jnp.sum(tb)
    return jnp.concatenate(blocks, axis=0)


def _cumsum_body(ds_ref, dc_ref, out_ref):
    dsum = ds_ref[0, :4096] + ds_ref[1, :4096]
    dcnt = dc_ref[0, :4096] + dc_ref[1, :4096]
    r = lax.broadcasted_iota(jnp.int32, (128, 128), 0)
    col = lax.broadcasted_iota(jnp.int32, (128, 128), 1)
    t128 = (r <= col).astype(jnp.float32)
    s128 = (col < r).astype(jnp.float32)
    sums = _cumsum_rowmajor(dsum, t128, s128)
    counts = _cumsum_rowmajor(dcnt, t128, s128)
    pos = counts > 0.0
    bp = jnp.where(pos, sums / jnp.where(pos, counts, 1.0), 0.0)
    out_ref[...] = _cumsum_rowmajor(bp, t128, s128)


def _cumsum_tc(dsum, dcnt):
    return pl.pallas_call(
        _cumsum_body,
        out_shape=jax.ShapeDtypeStruct((4096, 128), jnp.float32),
    )(dsum, dcnt)


@functools.partial(
    pl.kernel,
    out_type=jax.ShapeDtypeStruct((NP,), jnp.float32),
    mesh=_mesh,
    scratch_types=[
        pltpu.VMEM((CHUNK,), jnp.int32),
        pltpu.VMEM((CHUNK,), jnp.int32),
        pltpu.VMEM((CHUNK,), jnp.int32),
        pltpu.VMEM((CHUNK,), jnp.float32),
        pltpu.SemaphoreType.DMA,
    ],
    compiler_params=_sc_params,
)
def _gather_out(sp_hbm, tl_hbm, cum_hbm, out_hbm, sp_v, tl_v, idx_v, res_v, sem):
    c = lax.axis_index("c")
    s = lax.axis_index("s")
    wid = s * 2 + c

    for k in range(CHUNKS_PER_TILE):
        gbase = (wid * CHUNKS_PER_TILE + k) * CHUNK
        pltpu.sync_copy(sp_hbm.at[pl.ds(gbase, CHUNK)], sp_v)
        pltpu.sync_copy(tl_hbm.at[pl.ds(gbase, CHUNK)], tl_v)

        def _group(g, _):
            end = sp_v[pl.ds(g * 16, 16)] + tl_v[pl.ds(g * 16, 16)] - 1
            idx_v[pl.ds(g * 16, 16)] = end
            return 0
        lax.fori_loop(0, GROUPS, _group, 0)

        pltpu.async_copy(cum_hbm.at[idx_v], res_v, sem).wait()
        pltpu.sync_copy(res_v, out_hbm.at[pl.ds(gbase, CHUNK)])


def kernel(start_pos, token_id, token_len, non_dominated, params):
    pad = NP - N
    sp = jnp.pad(start_pos, (0, pad))
    ti = jnp.pad(token_id, (0, pad))
    tl = jnp.pad(token_len, (0, pad), constant_values=1)
    nd = jnp.pad(non_dominated, (0, pad)).astype(jnp.int32)
    par = jnp.pad(params, (0, VP - V))

    dsum, dcnt = _scatter_diff(sp, ti, tl, nd, par)
    cum = _cumsum_tc(dsum.reshape(2, PADL // 128, 128),
                     dcnt.reshape(2, PADL // 128, 128))
    out = _gather_out(sp, tl, cum.reshape(L))
    return out[:N]


# speedup vs baseline: 1.1226x; 1.1226x over previous
import functools

import jax
import jax.numpy as jnp
from jax import lax
from jax.experimental import pallas as pl
from jax.experimental.pallas import tpu as pltpu
from jax.experimental.pallas import tpu_sc as plsc

N = 1_000_000
V = 100_000
L = 524_288
MAXLEN = 8

VP = 100_096
NW = 32
CHUNK = 4096
NP = 1 << 20
CHUNKS_PER_TILE = NP // (NW * CHUNK)
GROUPS = CHUNK // 16
PADL = L + 256
TRASH = L
ZSLICE = L // 16
WSLICE = PADL // 16

_mesh = plsc.VectorSubcoreMesh(core_axis_name="c", subcore_axis_name="s")
_sc_params = pltpu.CompilerParams(use_tc_tiling_on_sc=False,
                                  needs_layout_passes=False)


@functools.partial(
    pl.kernel,
    out_type=(jax.ShapeDtypeStruct((2, PADL), jnp.float32),
              jax.ShapeDtypeStruct((2, PADL), jnp.float32)),
    mesh=_mesh,
    scratch_types=[
        pltpu.VMEM((CHUNK,), jnp.int32),
        pltpu.VMEM((CHUNK,), jnp.int32),
        pltpu.VMEM((CHUNK,), jnp.int32),
        pltpu.VMEM((CHUNK,), jnp.int32),
        pltpu.VMEM((CHUNK,), jnp.float32),
        pltpu.VMEM((CHUNK,), jnp.float32),
        pltpu.VMEM((CHUNK,), jnp.int32),
        pltpu.VMEM((CHUNK,), jnp.int32),
        pltpu.VMEM((CHUNK,), jnp.float32),
        pltpu.VMEM((CHUNK,), jnp.float32),
        pltpu.VMEM((2 * CHUNK,), jnp.float32),
        pltpu.VMEM_SHARED((PADL,), jnp.float32),
        pltpu.VMEM_SHARED((PADL,), jnp.float32),
        pltpu.VMEM_SHARED((VP,), jnp.float32),
        pltpu.SemaphoreType.DMA,
    ],
    compiler_params=_sc_params,
)
def _scatter_diff(sp_hbm, ti_hbm, tl_hbm, nd_hbm, par_hbm, osum_hbm, ocnt_hbm,
                  sp_v, ti_v, tl_v, nd_v, vals_v, vneg_v, ia_v, ib_v,
                  cpos_v, cneg_v, zf_v, asum_sh, acnt_sh, par_sh, sem):
    c = lax.axis_index("c")
    s = lax.axis_index("s")
    wid = s * 2 + c

    ones_f = jnp.full((16,), 1.0, jnp.float32)
    zero_f = jnp.zeros((16,), jnp.float32)

    def _fill_cnt(i, _):
        cpos_v[pl.ds(i * 16, 16)] = ones_f
        cneg_v[pl.ds(i * 16, 16)] = -ones_f
        return 0
    lax.fori_loop(0, CHUNK // 16, _fill_cnt, 0)

    def _fill_zf(i, _):
        zf_v[pl.ds(i * 16, 16)] = zero_f
        return 0
    lax.fori_loop(0, 2 * CHUNK // 16, _fill_zf, 0)

    for k in range(ZSLICE // (2 * CHUNK)):
        pltpu.sync_copy(zf_v, asum_sh.at[pl.ds(s * ZSLICE + k * 2 * CHUNK,
                                               2 * CHUNK)])
    for k in range(ZSLICE // (2 * CHUNK)):
        pltpu.sync_copy(zf_v, acnt_sh.at[pl.ds(s * ZSLICE + k * 2 * CHUNK,
                                               2 * CHUNK)])
    @pl.when(s == 0)
    def _():
        pltpu.sync_copy(zf_v.at[pl.ds(0, 256)], asum_sh.at[pl.ds(L, 256)])
        pltpu.sync_copy(zf_v.at[pl.ds(0, 256)], acnt_sh.at[pl.ds(L, 256)])

    pltpu.sync_copy(par_hbm.at[pl.ds(s * (VP // 16), VP // 16)],
                    par_sh.at[pl.ds(s * (VP // 16), VP // 16)])
    plsc.subcore_barrier()

    for k in range(CHUNKS_PER_TILE):
        gbase = (wid * CHUNKS_PER_TILE + k) * CHUNK
        pltpu.sync_copy(sp_hbm.at[pl.ds(gbase, CHUNK)], sp_v)
        pltpu.sync_copy(ti_hbm.at[pl.ds(gbase, CHUNK)], ti_v)
        pltpu.sync_copy(tl_hbm.at[pl.ds(gbase, CHUNK)], tl_v)
        pltpu.sync_copy(nd_hbm.at[pl.ds(gbase, CHUNK)], nd_v)
        pltpu.async_copy(par_sh.at[ti_v], vals_v, sem).wait()

        def _group(g, _):
            st = sp_v[pl.ds(g * 16, 16)]
            ln = tl_v[pl.ds(g * 16, 16)]
            nd = nd_v[pl.ds(g * 16, 16)] > 0
            val = vals_v[pl.ds(g * 16, 16)]
            ia_v[pl.ds(g * 16, 16)] = jnp.where(nd, st, TRASH)
            ib_v[pl.ds(g * 16, 16)] = jnp.where(nd, st + ln, TRASH)
            vneg_v[pl.ds(g * 16, 16)] = -val
            return 0
        lax.fori_loop(0, GROUPS, _group, 0)

        pltpu.sync_copy(vals_v, asum_sh.at[ia_v], add=True)
        pltpu.sync_copy(vneg_v, asum_sh.at[ib_v], add=True)
        pltpu.sync_copy(cpos_v, acnt_sh.at[ia_v], add=True)
        pltpu.sync_copy(cneg_v, acnt_sh.at[ib_v], add=True)

    plsc.subcore_barrier()
    pltpu.sync_copy(asum_sh.at[pl.ds(s * WSLICE, WSLICE)],
                    osum_hbm.at[c, pl.ds(s * WSLICE, WSLICE)])
    pltpu.sync_copy(acnt_sh.at[pl.ds(s * WSLICE, WSLICE)],
                    ocnt_hbm.at[c, pl.ds(s * WSLICE, WSLICE)])


def _mm(a, b):
    return lax.dot_general(a, b, (((1,), (0,)), ((), ())),
                           precision=lax.Precision.HIGHEST,
                           preferred_element_type=jnp.float32)


def _cumsum_rowmajor(x, t128, s128):
    y = _mm(x, t128)
    blocks = []
    run = jnp.float32(0.0)
    for b in range(32):
        yb = y[b * 128:(b + 1) * 128, :]
        tb = yb[:, 127:128]
        eb = _mm(s128, tb)
        blocks.append(yb + (eb + run))
        run = run + jnp.sum(tb)
    return jnp.concatenate(blocks, axis=0)


def _cumsum_body(ds_ref, dc_ref, out_ref):
    dsum = ds_ref[0, :4096] + ds_ref[1, :4096]
    dcnt = dc_ref[0, :4096] + dc_ref[1, :4096]
    r = lax.broadcasted_iota(jnp.int32, (128, 128), 0)
    col = lax.broadcasted_iota(jnp.int32, (128, 128), 1)
    t128 = (r <= col).astype(jnp.float32)
    s128 = (col < r).astype(jnp.float32)
    sums = _cumsum_rowmajor(dsum, t128, s128)
    counts = _cumsum_rowmajor(dcnt, t128, s128)
    pos = counts > 0.0
    bp = jnp.where(pos, sums / jnp.where(pos, counts, 1.0), 0.0)
    out_ref[...] = _cumsum_rowmajor(bp, t128, s128)


def _cumsum_tc(dsum, dcnt):
    return pl.pallas_call(
        _cumsum_body,
        out_shape=jax.ShapeDtypeStruct((4096, 128), jnp.float32),
    )(dsum, dcnt)


@functools.partial(
    pl.kernel,
    out_type=jax.ShapeDtypeStruct((NP,), jnp.float32),
    mesh=_mesh,
    scratch_types=[
        pltpu.VMEM((CHUNK,), jnp.int32),
        pltpu.VMEM((CHUNK,), jnp.int32),
        pltpu.VMEM((CHUNK,), jnp.int32),
        pltpu.VMEM((CHUNK,), jnp.float32),
        pltpu.VMEM_SHARED((L,), jnp.float32),
        pltpu.SemaphoreType.DMA,
    ],
    compiler_params=_sc_params,
)
def _gather_out(sp_hbm, tl_hbm, cum_hbm, out_hbm, sp_v, tl_v, idx_v, res_v,
                cum_sh, sem):
    c = lax.axis_index("c")
    s = lax.axis_index("s")
    wid = s * 2 + c

    pltpu.sync_copy(cum_hbm.at[pl.ds(s * (L // 16), L // 16)],
                    cum_sh.at[pl.ds(s * (L // 16), L // 16)])
    plsc.subcore_barrier()

    for k in range(CHUNKS_PER_TILE):
        gbase = (wid * CHUNKS_PER_TILE + k) * CHUNK
        pltpu.sync_copy(sp_hbm.at[pl.ds(gbase, CHUNK)], sp_v)
        pltpu.sync_copy(tl_hbm.at[pl.ds(gbase, CHUNK)], tl_v)

        def _group(g, _):
            end = sp_v[pl.ds(g * 16, 16)] + tl_v[pl.ds(g * 16, 16)] - 1
            idx_v[pl.ds(g * 16, 16)] = end
            return 0
        lax.fori_loop(0, GROUPS, _group, 0)

        pltpu.async_copy(cum_sh.at[idx_v], res_v, sem).wait()
        pltpu.sync_copy(res_v, out_hbm.at[pl.ds(gbase, CHUNK)])


def kernel(start_pos, token_id, token_len, non_dominated, params):
    pad = NP - N
    sp = jnp.pad(start_pos, (0, pad))
    ti = jnp.pad(token_id, (0, pad))
    tl = jnp.pad(token_len, (0, pad), constant_values=1)
    nd = jnp.pad(non_dominated, (0, pad)).astype(jnp.int32)
    par = jnp.pad(params, (0, VP - V))

    dsum, dcnt = _scatter_diff(sp, ti, tl, nd, par)
    cum = _cumsum_tc(dsum.reshape(2, PADL // 128, 128),
                     dcnt.reshape(2, PADL // 128, 128))
    out = _gather_out(sp, tl, cum.reshape(L))
    return out[:N]


# speedup vs baseline: 8.7186x; 7.7663x over previous
import functools

import jax
import jax.numpy as jnp
from jax import lax
from jax.experimental import pallas as pl
from jax.experimental.pallas import tpu as pltpu
from jax.experimental.pallas import tpu_sc as plsc

N = 1_000_000
V = 100_000
L = 524_288
MAXLEN = 8

VP = 100_096
NW = 32
CHUNK = 4096
NP = 1 << 20
CHUNKS_PER_TILE = NP // (NW * CHUNK)
GROUPS = CHUNK // 16
PADL = L + 256
TRASH = L
ZSLICE = L // 16
WSLICE = PADL // 16

_mesh = plsc.VectorSubcoreMesh(core_axis_name="c", subcore_axis_name="s")
_sc_params = pltpu.CompilerParams(use_tc_tiling_on_sc=False,
                                  needs_layout_passes=False)


@functools.partial(
    pl.kernel,
    out_type=(jax.ShapeDtypeStruct((2, PADL), jnp.float32),
              jax.ShapeDtypeStruct((2, PADL), jnp.float32)),
    mesh=_mesh,
    scratch_types=[
        pltpu.VMEM((CHUNK,), jnp.int32),
        pltpu.VMEM((CHUNK,), jnp.int32),
        pltpu.VMEM((CHUNK,), jnp.int32),
        pltpu.VMEM((CHUNK,), jnp.int32),
        pltpu.VMEM((CHUNK + 256,), jnp.int32),
        pltpu.VMEM((CHUNK + 256,), jnp.int32),
        pltpu.VMEM((CHUNK + 256,), jnp.int32),
        pltpu.VMEM((CHUNK + 256,), jnp.float32),
        pltpu.VMEM((CHUNK + 256,), jnp.float32),
        pltpu.VMEM((CHUNK + 256,), jnp.float32),
        pltpu.VMEM((CHUNK + 256,), jnp.float32),
        pltpu.VMEM((2 * CHUNK,), jnp.float32),
        pltpu.VMEM_SHARED((PADL,), jnp.float32),
        pltpu.VMEM_SHARED((PADL,), jnp.float32),
        pltpu.VMEM_SHARED((VP,), jnp.float32),
        pltpu.SemaphoreType.DMA,
    ],
    compiler_params=_sc_params,
)
def _scatter_diff(sp_hbm, ti_hbm, tl_hbm, nd_hbm, par_hbm, osum_hbm, ocnt_hbm,
                  sp_v, ti_v, tl_v, nd_v, tic_v, ia_v, ib_v, vals_v, vneg_v,
                  cpos_v, cneg_v, zf_v, asum_sh, acnt_sh, par_sh, sem):
    c = lax.axis_index("c")
    s = lax.axis_index("s")
    wid = s * 2 + c

    ones_f = jnp.full((16,), 1.0, jnp.float32)
    zero_f = jnp.zeros((16,), jnp.float32)
    zero_i = jnp.zeros((16,), jnp.int32)

    def _fill_cnt(i, _):
        cpos_v[pl.ds(i * 16, 16)] = ones_f
        cneg_v[pl.ds(i * 16, 16)] = -ones_f
        tic_v[pl.ds(i * 16, 16)] = zero_i
        return 0
    lax.fori_loop(0, (CHUNK + 256) // 16, _fill_cnt, 0)

    def _fill_zf(i, _):
        zf_v[pl.ds(i * 16, 16)] = zero_f
        return 0
    lax.fori_loop(0, 2 * CHUNK // 16, _fill_zf, 0)

    for k in range(ZSLICE // (2 * CHUNK)):
        pltpu.sync_copy(zf_v, asum_sh.at[pl.ds(s * ZSLICE + k * 2 * CHUNK,
                                               2 * CHUNK)])
    for k in range(ZSLICE // (2 * CHUNK)):
        pltpu.sync_copy(zf_v, acnt_sh.at[pl.ds(s * ZSLICE + k * 2 * CHUNK,
                                               2 * CHUNK)])
    @pl.when(s == 0)
    def _():
        pltpu.sync_copy(zf_v.at[pl.ds(0, 256)], asum_sh.at[pl.ds(L, 256)])
        pltpu.sync_copy(zf_v.at[pl.ds(0, 256)], acnt_sh.at[pl.ds(L, 256)])

    pltpu.sync_copy(par_hbm.at[pl.ds(s * (VP // 16), VP // 16)],
                    par_sh.at[pl.ds(s * (VP // 16), VP // 16)])
    plsc.subcore_barrier()

    for k in range(CHUNKS_PER_TILE):
        gbase = (wid * CHUNKS_PER_TILE + k) * CHUNK
        pltpu.sync_copy(sp_hbm.at[pl.ds(gbase, CHUNK)], sp_v)
        pltpu.sync_copy(ti_hbm.at[pl.ds(gbase, CHUNK)], ti_v)
        pltpu.sync_copy(tl_hbm.at[pl.ds(gbase, CHUNK)], tl_v)
        pltpu.sync_copy(nd_hbm.at[pl.ds(gbase, CHUNK)], nd_v)

        def _group(g, cnt):
            st = sp_v[pl.ds(g * 16, 16)]
            ln = tl_v[pl.ds(g * 16, 16)]
            nd = nd_v[pl.ds(g * 16, 16)] > 0
            tid = ti_v[pl.ds(g * 16, 16)]
            plsc.store_compressed(ia_v.at[pl.ds(cnt, 16)], st, mask=nd)
            plsc.store_compressed(ib_v.at[pl.ds(cnt, 16)], st + ln, mask=nd)
            plsc.store_compressed(tic_v.at[pl.ds(cnt, 16)], tid, mask=nd)
            return cnt + plsc.all_reduce_population_count(nd)[0]
        cnt = lax.fori_loop(0, GROUPS, _group, jnp.int32(0))

        trash_g = jnp.full((16,), TRASH, jnp.int32)
        for j in range(9):
            ia_v[pl.ds(cnt + j * 16, 16)] = trash_g
            ib_v[pl.ds(cnt + j * 16, 16)] = trash_g

        nstr = (cnt + 127) // 128

        def _pgath(j, _):
            sl = pl.ds(j * 128, 128)
            pltpu.async_copy(par_sh.at[tic_v.at[sl]], vals_v.at[sl], sem).wait()
            return 0
        lax.fori_loop(0, nstr, _pgath, 0)

        def _neg(g, _):
            vneg_v[pl.ds(g * 16, 16)] = -vals_v[pl.ds(g * 16, 16)]
            return 0
        lax.fori_loop(0, (cnt + 143) // 16, _neg, 0)

        def _scat(j, _):
            sl = pl.ds(j * 128, 128)
            pltpu.sync_copy(vals_v.at[sl], asum_sh.at[ia_v.at[sl]], add=True)
            pltpu.sync_copy(vneg_v.at[sl], asum_sh.at[ib_v.at[sl]], add=True)
            pltpu.sync_copy(cpos_v.at[sl], acnt_sh.at[ia_v.at[sl]], add=True)
            pltpu.sync_copy(cneg_v.at[sl], acnt_sh.at[ib_v.at[sl]], add=True)
            return 0
        lax.fori_loop(0, nstr, _scat, 0)

    plsc.subcore_barrier()
    pltpu.sync_copy(asum_sh.at[pl.ds(s * WSLICE, WSLICE)],
                    osum_hbm.at[c, pl.ds(s * WSLICE, WSLICE)])
    pltpu.sync_copy(acnt_sh.at[pl.ds(s * WSLICE, WSLICE)],
                    ocnt_hbm.at[c, pl.ds(s * WSLICE, WSLICE)])


def _mm(a, b):
    return lax.dot_general(a, b, (((1,), (0,)), ((), ())),
                           precision=lax.Precision.HIGHEST,
                           preferred_element_type=jnp.float32)


def _cumsum_rowmajor(x, t128, s128):
    y = _mm(x, t128)
    blocks = []
    run = jnp.float32(0.0)
    for b in range(32):
        yb = y[b * 128:(b + 1) * 128, :]
        tb = yb[:, 127:128]
        eb = _mm(s128, tb)
        blocks.append(yb + (eb + run))
        run = run + jnp.sum(tb)
    return jnp.concatenate(blocks, axis=0)


def _cumsum_body(ds_ref, dc_ref, out_ref):
    dsum = ds_ref[0, :4096] + ds_ref[1, :4096]
    dcnt = dc_ref[0, :4096] + dc_ref[1, :4096]
    r = lax.broadcasted_iota(jnp.int32, (128, 128), 0)
    col = lax.broadcasted_iota(jnp.int32, (128, 128), 1)
    t128 = (r <= col).astype(jnp.float32)
    s128 = (col < r).astype(jnp.float32)
    sums = _cumsum_rowmajor(dsum, t128, s128)
    counts = _cumsum_rowmajor(dcnt, t128, s128)
    pos = counts > 0.0
    bp = jnp.where(pos, sums / jnp.where(pos, counts, 1.0), 0.0)
    out_ref[...] = _cumsum_rowmajor(bp, t128, s128)


def _cumsum_tc(dsum, dcnt):
    return pl.pallas_call(
        _cumsum_body,
        out_shape=jax.ShapeDtypeStruct((4096, 128), jnp.float32),
    )(dsum, dcnt)


@functools.partial(
    pl.kernel,
    out_type=jax.ShapeDtypeStruct((NP,), jnp.float32),
    mesh=_mesh,
    scratch_types=[
        pltpu.VMEM((CHUNK,), jnp.int32),
        pltpu.VMEM((CHUNK,), jnp.int32),
        pltpu.VMEM((CHUNK,), jnp.int32),
        pltpu.VMEM((CHUNK,), jnp.float32),
        pltpu.VMEM_SHARED((L,), jnp.float32),
        pltpu.SemaphoreType.DMA,
    ],
    compiler_params=_sc_params,
)
def _gather_out(sp_hbm, tl_hbm, cum_hbm, out_hbm, sp_v, tl_v, idx_v, res_v,
                cum_sh, sem):
    c = lax.axis_index("c")
    s = lax.axis_index("s")
    wid = s * 2 + c

    pltpu.sync_copy(cum_hbm.at[pl.ds(s * (L // 16), L // 16)],
                    cum_sh.at[pl.ds(s * (L // 16), L // 16)])
    plsc.subcore_barrier()

    for k in range(CHUNKS_PER_TILE):
        gbase = (wid * CHUNKS_PER_TILE + k) * CHUNK
        pltpu.sync_copy(sp_hbm.at[pl.ds(gbase, CHUNK)], sp_v)
        pltpu.sync_copy(tl_hbm.at[pl.ds(gbase, CHUNK)], tl_v)

        def _group(g, _):
            end = sp_v[pl.ds(g * 16, 16)] + tl_v[pl.ds(g * 16, 16)] - 1
            idx_v[pl.ds(g * 16, 16)] = end
            return 0
        lax.fori_loop(0, GROUPS, _group, 0)

        pltpu.async_copy(cum_sh.at[idx_v], res_v, sem).wait()
        pltpu.sync_copy(res_v, out_hbm.at[pl.ds(gbase, CHUNK)])


def kernel(start_pos, token_id, token_len, non_dominated, params):
    pad = NP - N
    sp = jnp.pad(start_pos, (0, pad))
    ti = jnp.pad(token_id, (0, pad))
    tl = jnp.pad(token_len, (0, pad), constant_values=1)
    nd = jnp.pad(non_dominated, (0, pad)).astype(jnp.int32)
    par = jnp.pad(params, (0, VP - V))

    dsum, dcnt = _scatter_diff(sp, ti, tl, nd, par)
    cum = _cumsum_tc(dsum.reshape(2, PADL // 128, 128),
                     dcnt.reshape(2, PADL // 128, 128))
    out = _gather_out(sp, tl, cum.reshape(L))
    return out[:N]


# speedup vs baseline: 9.7932x; 1.1233x over previous
import functools

import jax
import jax.numpy as jnp
from jax import lax
from jax.experimental import pallas as pl
from jax.experimental.pallas import tpu as pltpu
from jax.experimental.pallas import tpu_sc as plsc

N = 1_000_000
V = 100_000
L = 524_288
MAXLEN = 8

VP = 100_096
NW = 32
CHUNK = 4096
NP = 1 << 20
CHUNKS_PER_TILE = NP // (NW * CHUNK)
GROUPS = CHUNK // 16
PADL = L + 256
TRASH = L
ZSLICE = L // 16
WSLICE = PADL // 16

_mesh = plsc.VectorSubcoreMesh(core_axis_name="c", subcore_axis_name="s")
_sc_params = pltpu.CompilerParams(use_tc_tiling_on_sc=False,
                                  needs_layout_passes=False)


@functools.partial(
    pl.kernel,
    out_type=(jax.ShapeDtypeStruct((2, PADL), jnp.float32),
              jax.ShapeDtypeStruct((2, PADL), jnp.float32)),
    mesh=_mesh,
    scratch_types=[
        pltpu.VMEM((CHUNK,), jnp.int32),
        pltpu.VMEM((CHUNK,), jnp.int32),
        pltpu.VMEM((CHUNK,), jnp.int32),
        pltpu.VMEM((CHUNK,), jnp.int32),
        pltpu.VMEM((CHUNK + 256,), jnp.int32),
        pltpu.VMEM((CHUNK + 256,), jnp.int32),
        pltpu.VMEM((CHUNK + 256,), jnp.int32),
        pltpu.VMEM((CHUNK + 256,), jnp.float32),
        pltpu.VMEM((CHUNK + 256,), jnp.float32),
        pltpu.VMEM((CHUNK + 256,), jnp.float32),
        pltpu.VMEM((CHUNK + 256,), jnp.float32),
        pltpu.VMEM((2 * CHUNK,), jnp.float32),
        pltpu.VMEM_SHARED((PADL,), jnp.float32),
        pltpu.VMEM_SHARED((PADL,), jnp.float32),
        pltpu.VMEM_SHARED((VP,), jnp.float32),
        pltpu.SemaphoreType.DMA,
    ],
    compiler_params=_sc_params,
)
def _scatter_diff(sp_hbm, ti_hbm, tl_hbm, nd_hbm, par_hbm, osum_hbm, ocnt_hbm,
                  sp_v, ti_v, tl_v, nd_v, tic_v, ia_v, ib_v, vals_v, vneg_v,
                  cpos_v, cneg_v, zf_v, asum_sh, acnt_sh, par_sh, sem):
    c = lax.axis_index("c")
    s = lax.axis_index("s")
    wid = s * 2 + c

    ones_f = jnp.full((16,), 1.0, jnp.float32)
    zero_f = jnp.zeros((16,), jnp.float32)
    zero_i = jnp.zeros((16,), jnp.int32)

    def _fill_cnt(i, _):
        cpos_v[pl.ds(i * 16, 16)] = ones_f
        cneg_v[pl.ds(i * 16, 16)] = -ones_f
        tic_v[pl.ds(i * 16, 16)] = zero_i
        return 0
    lax.fori_loop(0, (CHUNK + 256) // 16, _fill_cnt, 0)

    def _fill_zf(i, _):
        zf_v[pl.ds(i * 16, 16)] = zero_f
        return 0
    lax.fori_loop(0, 2 * CHUNK // 16, _fill_zf, 0)

    for k in range(ZSLICE // (2 * CHUNK)):
        pltpu.sync_copy(zf_v, asum_sh.at[pl.ds(s * ZSLICE + k * 2 * CHUNK,
                                               2 * CHUNK)])
    for k in range(ZSLICE // (2 * CHUNK)):
        pltpu.sync_copy(zf_v, acnt_sh.at[pl.ds(s * ZSLICE + k * 2 * CHUNK,
                                               2 * CHUNK)])
    @pl.when(s == 0)
    def _():
        pltpu.sync_copy(zf_v.at[pl.ds(0, 256)], asum_sh.at[pl.ds(L, 256)])
        pltpu.sync_copy(zf_v.at[pl.ds(0, 256)], acnt_sh.at[pl.ds(L, 256)])

    pltpu.sync_copy(par_hbm.at[pl.ds(s * (VP // 16), VP // 16)],
                    par_sh.at[pl.ds(s * (VP // 16), VP // 16)])
    plsc.subcore_barrier()

    for k in range(CHUNKS_PER_TILE):
        gbase = (wid * CHUNKS_PER_TILE + k) * CHUNK
        d0 = pltpu.async_copy(sp_hbm.at[pl.ds(gbase, CHUNK)], sp_v, sem)
        d1 = pltpu.async_copy(ti_hbm.at[pl.ds(gbase, CHUNK)], ti_v, sem)
        d2 = pltpu.async_copy(tl_hbm.at[pl.ds(gbase, CHUNK)], tl_v, sem)
        d3 = pltpu.async_copy(nd_hbm.at[pl.ds(gbase, CHUNK)], nd_v, sem)
        d0.wait()
        d1.wait()
        d2.wait()
        d3.wait()

        def _group(g, cnt):
            st = sp_v[pl.ds(g * 16, 16)]
            ln = tl_v[pl.ds(g * 16, 16)]
            nd = nd_v[pl.ds(g * 16, 16)] > 0
            tid = ti_v[pl.ds(g * 16, 16)]
            plsc.store_compressed(ia_v.at[pl.ds(cnt, 16)], st, mask=nd)
            plsc.store_compressed(ib_v.at[pl.ds(cnt, 16)], st + ln, mask=nd)
            plsc.store_compressed(tic_v.at[pl.ds(cnt, 16)], tid, mask=nd)
            return cnt + plsc.all_reduce_population_count(nd)[0]
        cnt = lax.fori_loop(0, GROUPS, _group, jnp.int32(0))

        trash_g = jnp.full((16,), TRASH, jnp.int32)
        for j in range(9):
            ia_v[pl.ds(cnt + j * 16, 16)] = trash_g
            ib_v[pl.ds(cnt + j * 16, 16)] = trash_g

        nstr = (cnt + 127) // 128

        def _pgath(j, _):
            sl = pl.ds(j * 128, 128)
            pltpu.async_copy(par_sh.at[tic_v.at[sl]], vals_v.at[sl], sem).wait()
            return 0
        lax.fori_loop(0, nstr, _pgath, 0)

        def _neg(g, _):
            vneg_v[pl.ds(g * 16, 16)] = -vals_v[pl.ds(g * 16, 16)]
            return 0
        lax.fori_loop(0, (cnt + 143) // 16, _neg, 0)

        def _scat(j, _):
            sl = pl.ds(j * 128, 128)
            e0 = pltpu.async_copy(vals_v.at[sl], asum_sh.at[ia_v.at[sl]],
                                  sem, add=True)
            e1 = pltpu.async_copy(vneg_v.at[sl], asum_sh.at[ib_v.at[sl]],
                                  sem, add=True)
            e2 = pltpu.async_copy(cpos_v.at[sl], acnt_sh.at[ia_v.at[sl]],
                                  sem, add=True)
            e3 = pltpu.async_copy(cneg_v.at[sl], acnt_sh.at[ib_v.at[sl]],
                                  sem, add=True)
            e0.wait()
            e1.wait()
            e2.wait()
            e3.wait()
            return 0
        lax.fori_loop(0, nstr, _scat, 0)

    plsc.subcore_barrier()
    pltpu.sync_copy(asum_sh.at[pl.ds(s * WSLICE, WSLICE)],
                    osum_hbm.at[c, pl.ds(s * WSLICE, WSLICE)])
    pltpu.sync_copy(acnt_sh.at[pl.ds(s * WSLICE, WSLICE)],
                    ocnt_hbm.at[c, pl.ds(s * WSLICE, WSLICE)])


def _mm(a, b):
    return lax.dot_general(a, b, (((1,), (0,)), ((), ())),
                           precision=lax.Precision.HIGHEST,
                           preferred_element_type=jnp.float32)


def _cumsum_rowmajor(x, t128, s128):
    y = _mm(x, t128)
    blocks = []
    run = jnp.float32(0.0)
    for b in range(32):
        yb = y[b * 128:(b + 1) * 128, :]
        tb = yb[:, 127:128]
        eb = _mm(s128, tb)
        blocks.append(yb + (eb + run))
        run = run + jnp.sum(tb)
    return jnp.concatenate(blocks, axis=0)


def _cumsum_body(ds_ref, dc_ref, out_ref):
    dsum = ds_ref[0, :4096] + ds_ref[1, :4096]
    dcnt = dc_ref[0, :4096] + dc_ref[1, :4096]
    r = lax.broadcasted_iota(jnp.int32, (128, 128), 0)
    col = lax.broadcasted_iota(jnp.int32, (128, 128), 1)
    t128 = (r <= col).astype(jnp.float32)
    s128 = (col < r).astype(jnp.float32)
    sums = _cumsum_rowmajor(dsum, t128, s128)
    counts = _cumsum_rowmajor(dcnt, t128, s128)
    pos = counts > 0.0
    bp = jnp.where(pos, sums / jnp.where(pos, counts, 1.0), 0.0)
    out_ref[...] = _cumsum_rowmajor(bp, t128, s128)


def _cumsum_tc(dsum, dcnt):
    return pl.pallas_call(
        _cumsum_body,
        out_shape=jax.ShapeDtypeStruct((4096, 128), jnp.float32),
    )(dsum, dcnt)


@functools.partial(
    pl.kernel,
    out_type=jax.ShapeDtypeStruct((NP,), jnp.float32),
    mesh=_mesh,
    scratch_types=[
        pltpu.VMEM((CHUNK,), jnp.int32),
        pltpu.VMEM((CHUNK,), jnp.int32),
        pltpu.VMEM((CHUNK,), jnp.int32),
        pltpu.VMEM((CHUNK,), jnp.float32),
        pltpu.VMEM_SHARED((L,), jnp.float32),
        pltpu.SemaphoreType.DMA,
    ],
    compiler_params=_sc_params,
)
def _gather_out(sp_hbm, tl_hbm, cum_hbm, out_hbm, sp_v, tl_v, idx_v, res_v,
                cum_sh, sem):
    c = lax.axis_index("c")
    s = lax.axis_index("s")
    wid = s * 2 + c

    pltpu.sync_copy(cum_hbm.at[pl.ds(s * (L // 16), L // 16)],
                    cum_sh.at[pl.ds(s * (L // 16), L // 16)])
    plsc.subcore_barrier()

    for k in range(CHUNKS_PER_TILE):
        gbase = (wid * CHUNKS_PER_TILE + k) * CHUNK
        d0 = pltpu.async_copy(sp_hbm.at[pl.ds(gbase, CHUNK)], sp_v, sem)
        d1 = pltpu.async_copy(tl_hbm.at[pl.ds(gbase, CHUNK)], tl_v, sem)
        d0.wait()
        d1.wait()

        def _group(g, _):
            end = sp_v[pl.ds(g * 16, 16)] + tl_v[pl.ds(g * 16, 16)] - 1
            idx_v[pl.ds(g * 16, 16)] = end
            return 0
        lax.fori_loop(0, GROUPS, _group, 0)

        def _gath(j, _):
            g0 = pltpu.async_copy(cum_sh.at[idx_v.at[pl.ds(j * 512, 128)]],
                                  res_v.at[pl.ds(j * 512, 128)], sem)
            g1 = pltpu.async_copy(cum_sh.at[idx_v.at[pl.ds(j * 512 + 128, 128)]],
                                  res_v.at[pl.ds(j * 512 + 128, 128)], sem)
            g2 = pltpu.async_copy(cum_sh.at[idx_v.at[pl.ds(j * 512 + 256, 128)]],
                                  res_v.at[pl.ds(j * 512 + 256, 128)], sem)
            g3 = pltpu.async_copy(cum_sh.at[idx_v.at[pl.ds(j * 512 + 384, 128)]],
                                  res_v.at[pl.ds(j * 512 + 384, 128)], sem)
            g0.wait()
            g1.wait()
            g2.wait()
            g3.wait()
            return 0
        lax.fori_loop(0, CHUNK // 512, _gath, 0)
        pltpu.sync_copy(res_v, out_hbm.at[pl.ds(gbase, CHUNK)])


def kernel(start_pos, token_id, token_len, non_dominated, params):
    pad = NP - N
    sp = jnp.pad(start_pos, (0, pad))
    ti = jnp.pad(token_id, (0, pad))
    tl = jnp.pad(token_len, (0, pad), constant_values=1)
    nd = jnp.pad(non_dominated, (0, pad)).astype(jnp.int32)
    par = jnp.pad(params, (0, VP - V))

    dsum, dcnt = _scatter_diff(sp, ti, tl, nd, par)
    cum = _cumsum_tc(dsum.reshape(2, PADL // 128, 128),
                     dcnt.reshape(2, PADL // 128, 128))
    out = _gather_out(sp, tl, cum.reshape(L))
    return out[:N]
